# Initial kernel scaffold; baseline (speedup 1.0000x reference)
import jax
import jax.numpy as jnp
from jax.experimental import pallas as pl


def kernel(x, edge_index, edge_attr, ggnn_w, gru_w_ih, gru_w_hh, gru_b_ih, gru_b_hh, conv1_w, conv1_b, conv2_w, conv2_b, conv1c_w, conv1c_b, conv2c_w, conv2c_b, bn_g, bn_b, bnc_g, bnc_b, mlpy_w, mlpy_b, mlpz_w, mlpz_b):
    raise NotImplementedError("write your pallas kernel here")


# speedup vs baseline: 3.3958x; 3.3958x over previous
import functools

import jax
import jax.numpy as jnp
from jax import lax
from jax.experimental import pallas as pl
from jax.experimental.pallas import tpu as pltpu
from jax.experimental.pallas import tpu_sc as plsc

_T = 6
_NC = 2
_NS = 16


@functools.cache
def _make_sc_msg(n, d, e):
    epc = e // _NC
    epw = epc // _NS
    k = 80
    assert epw % k == 0
    nchunks = epw // k
    rpw = 624
    zr = 208
    tail = n - (_NS - 1) * rpw - 3 * zr
    assert 0 <= tail <= zr and tail % 8 == 0

    mesh = plsc.VectorSubcoreMesh(
        core_axis_name="c", subcore_axis_name="s", num_cores=_NC)

    @functools.partial(
        pl.kernel,
        out_type=jax.ShapeDtypeStruct((_NC * n, d), jnp.float32),
        mesh=mesh,
        scratch_types=[
            pltpu.VMEM((k,), jnp.int32),
            pltpu.VMEM((k,), jnp.int32),
            pltpu.VMEM((k,), jnp.float32),
            pltpu.VMEM((k, d), jnp.float32),
            pltpu.VMEM((zr, d), jnp.float32),
            pltpu.VMEM_SHARED((n, d), jnp.float32),
            pltpu.SemaphoreType.DMA,
        ],
    )
    def sc_msg(m_hbm, src_hbm, dst_hbm, ea_hbm, out_hbm,
               src_v, dst_v, ea_v, rows_v, zb_v, agg_sh, sem):
        c = lax.axis_index("c")
        s = lax.axis_index("s")

        zeros = jnp.zeros((16,), jnp.float32)

        def _zb(i, carry):
            for j in range(d // 16):
                zb_v[i, pl.ds(j * 16, 16)] = zeros
            return carry
        lax.fori_loop(0, zr, _zb, 0)
        for t in range(3):
            pltpu.sync_copy(zb_v, agg_sh.at[pl.ds(s * rpw + t * zr, zr)])

        @pl.when(s == _NS - 1)
        def _zero_tail():
            pltpu.sync_copy(zb_v.at[pl.ds(0, tail)],
                            agg_sh.at[pl.ds(_NS * rpw, tail)])
        plsc.subcore_barrier()

        ebase = c * epc + s * epw

        def _chunk(i, carry):
            off = ebase + i * k
            pltpu.sync_copy(src_hbm.at[pl.ds(off, k)], src_v)
            pltpu.sync_copy(dst_hbm.at[pl.ds(off, k)], dst_v)
            pltpu.sync_copy(ea_hbm.at[pl.ds(off, k)], ea_v)
            pltpu.async_copy(m_hbm.at[src_v], rows_v, sem).wait()

            for g in range(k // 16):
                ea16 = ea_v[pl.ds(g * 16, 16)]

                def _edge(ii, carry2, g=g, ea16=ea16):
                    ke = g * 16 + ii
                    scale = ea16.at[jnp.full((16,), ii, jnp.int32)].get(
                        mode='promise_in_bounds')
                    for j in range(d // 16):
                        rows_v[ke, pl.ds(j * 16, 16)] = (
                            rows_v[ke, pl.ds(j * 16, 16)] * scale)
                    return carry2
                lax.fori_loop(0, 16, _edge, 0)

            pltpu.sync_copy(rows_v, agg_sh.at[dst_v], add=True)
            return carry
        lax.fori_loop(0, nchunks, _chunk, 0)

        plsc.subcore_barrier()
        for t in range(3):
            r0 = s * rpw + t * zr
            pltpu.sync_copy(agg_sh.at[pl.ds(r0, zr)],
                            out_hbm.at[pl.ds(c * n + r0, zr)])

        @pl.when(s == _NS - 1)
        def _write_tail():
            r0 = _NS * rpw
            pltpu.sync_copy(agg_sh.at[pl.ds(r0, tail)],
                            out_hbm.at[pl.ds(c * n + r0, tail)])

    return sc_msg


def _mm_body(h_ref, w_ref, o_ref):
    o_ref[...] = jnp.dot(h_ref[...], w_ref[...],
                         preferred_element_type=jnp.float32)


def _matmul(h, w):
    n, d = h.shape
    dout = w.shape[1]
    blk = 2000
    grid = n // blk
    return pl.pallas_call(
        _mm_body,
        grid=(grid,),
        in_specs=[
            pl.BlockSpec((blk, d), lambda i: (i, 0)),
            pl.BlockSpec((d, dout), lambda i: (0, 0)),
        ],
        out_specs=pl.BlockSpec((blk, dout), lambda i: (i, 0)),
        out_shape=jax.ShapeDtypeStruct((n, dout), jnp.float32),
    )(h, w)


def _gru_body(a0_ref, a1_ref, h_ref, wih_ref, whh_ref, bih_ref, bhh_ref,
              wnext_ref, hn_ref, mn_ref):
    d = h_ref.shape[1]
    a = a0_ref[...] + a1_ref[...]
    h = h_ref[...]
    gi = jnp.dot(a, wih_ref[...], preferred_element_type=jnp.float32) \
        + bih_ref[...]
    gh = jnp.dot(h, whh_ref[...], preferred_element_type=jnp.float32) \
        + bhh_ref[...]
    r = jax.nn.sigmoid(gi[:, 0:d] + gh[:, 0:d])
    z = jax.nn.sigmoid(gi[:, d:2 * d] + gh[:, d:2 * d])
    nn = jnp.tanh(gi[:, 2 * d:3 * d] + r * gh[:, 2 * d:3 * d])
    hn = (1.0 - z) * nn + z * h
    hn_ref[...] = hn
    mn_ref[...] = jnp.dot(hn, wnext_ref[...],
                          preferred_element_type=jnp.float32)


def _gru(a0, a1, h, wih_t, whh_t, bih, bhh, wnext):
    n, d = h.shape
    blk = 2000
    grid = n // blk
    return pl.pallas_call(
        _gru_body,
        grid=(grid,),
        in_specs=[
            pl.BlockSpec((blk, d), lambda i: (i, 0)),
            pl.BlockSpec((blk, d), lambda i: (i, 0)),
            pl.BlockSpec((blk, d), lambda i: (i, 0)),
            pl.BlockSpec((d, 3 * d), lambda i: (0, 0)),
            pl.BlockSpec((d, 3 * d), lambda i: (0, 0)),
            pl.BlockSpec((1, 3 * d), lambda i: (0, 0)),
            pl.BlockSpec((1, 3 * d), lambda i: (0, 0)),
            pl.BlockSpec((d, d), lambda i: (0, 0)),
        ],
        out_specs=[
            pl.BlockSpec((blk, d), lambda i: (i, 0)),
            pl.BlockSpec((blk, d), lambda i: (i, 0)),
        ],
        out_shape=[
            jax.ShapeDtypeStruct((n, d), jnp.float32),
            jax.ShapeDtypeStruct((n, d), jnp.float32),
        ],
    )(a0, a1, h, wih_t, whh_t, bih, bhh, wnext)


def _h1y_body(x_ref, w0_ref, w1_ref, w2_ref, b_ref, g_ref, be_ref, y_ref):
    lc = y_ref.shape[0]
    c1 = (jnp.dot(x_ref[pl.ds(0, lc), :], w0_ref[...],
                  preferred_element_type=jnp.float32)
          + jnp.dot(x_ref[pl.ds(1, lc), :], w1_ref[...],
                    preferred_element_type=jnp.float32)
          + jnp.dot(x_ref[pl.ds(2, lc), :], w2_ref[...],
                    preferred_element_type=jnp.float32)
          + b_ref[...])
    mu = jnp.mean(c1, axis=0, keepdims=True)
    var = jnp.mean(c1 * c1, axis=0, keepdims=True) - mu * mu
    y = (c1 - mu) * lax.rsqrt(var + 1e-5) * g_ref[...] + be_ref[...]
    y_ref[...] = jnp.maximum(y, 0.0)


def _h1y(x, w0, w1, w2, b, g, be):
    l, d = x.shape
    ch = w0.shape[1]
    lc = l - 2
    return pl.pallas_call(
        _h1y_body,
        out_shape=jax.ShapeDtypeStruct((lc, ch), jnp.float32),
    )(x, w0, w1, w2, b, g, be)


def _h1z_body(h_ref, x_ref, wh0_ref, wh1_ref, wh2_ref,
              wx0_ref, wx1_ref, wx2_ref, b_ref, g_ref, be_ref, y_ref):
    lc = y_ref.shape[0]
    c1 = b_ref[...]
    for (src_ref, wrefs) in ((h_ref, (wh0_ref, wh1_ref, wh2_ref)),
                             (x_ref, (wx0_ref, wx1_ref, wx2_ref))):
        for t in range(3):
            c1 = c1 + jnp.dot(src_ref[pl.ds(t, lc), :], wrefs[t][...],
                              preferred_element_type=jnp.float32)
    mu = jnp.mean(c1, axis=0, keepdims=True)
    var = jnp.mean(c1 * c1, axis=0, keepdims=True) - mu * mu
    y = (c1 - mu) * lax.rsqrt(var + 1e-5) * g_ref[...] + be_ref[...]
    y_ref[...] = jnp.maximum(y, 0.0)


def _h1z(h, x, wh, wx, b, g, be):
    l, d = h.shape
    ch = wh[0].shape[1]
    lc = l - 2
    return pl.pallas_call(
        _h1z_body,
        out_shape=jax.ShapeDtypeStruct((lc, ch), jnp.float32),
    )(h, x, wh[0], wh[1], wh[2], wx[0], wx[1], wx[2], b, g, be)


def _h2_body(p_ref, w2_ref, b2_ref, g_ref, be_ref, y_ref):
    lo = y_ref.shape[0]
    v = p_ref[...]
    pm = jnp.maximum(v[:, 0, :], v[:, 1, :])
    ev = v[:, 0, :]
    out1 = jnp.maximum(pm[0:lo, :], ev[1:lo + 1, :])
    c2 = jnp.dot(out1, w2_ref[...],
                 preferred_element_type=jnp.float32) + b2_ref[...]
    mu = jnp.mean(c2, axis=0, keepdims=True)
    var = jnp.mean(c2 * c2, axis=0, keepdims=True) - mu * mu
    y = (c2 - mu) * lax.rsqrt(var + 1e-5) * g_ref[...] + be_ref[...]
    y_ref[...] = jnp.maximum(y, 0.0)


def _h2(p, w2, b2, g, be):
    lp, _, ch = p.shape
    lo = lp - 1
    return pl.pallas_call(
        _h2_body,
        out_shape=jax.ShapeDtypeStruct((lo, ch), jnp.float32),
    )(p, w2, b2, g, be)


def _h3_body(p_ref, mw_ref, mb_ref, o_ref):
    v = p_ref[...]
    pm = jnp.maximum(v[:, 0, :], v[:, 1, :])
    o_ref[...] = jnp.dot(pm, mw_ref[...],
                         preferred_element_type=jnp.float32) + mb_ref[...]


def _h3(p, mw, mb):
    lp = p.shape[0]
    return pl.pallas_call(
        _h3_body,
        out_shape=jax.ShapeDtypeStruct((lp, 1), jnp.float32),
    )(p, mw, mb)


def _h4_body(y_ref, z_ref, o_ref):
    prod = y_ref[...] * z_ref[...]
    m = jnp.sum(prod, axis=0, keepdims=True) / prod.shape[0]
    o_ref[...] = jax.nn.sigmoid(m)


def _h4(y, z):
    return pl.pallas_call(
        _h4_body,
        out_shape=jax.ShapeDtypeStruct((1, 1), jnp.float32),
    )(y, z)


def kernel(x, edge_index, edge_attr, ggnn_w, gru_w_ih, gru_w_hh, gru_b_ih,
           gru_b_hh, conv1_w, conv1_b, conv2_w, conv2_b, conv1c_w, conv1c_b,
           conv2c_w, conv2c_b, bn_g, bn_b, bnc_g, bnc_b, mlpy_w, mlpy_b,
           mlpz_w, mlpz_b):
    n, d = x.shape
    e = edge_attr.shape[0]
    c = conv1c_w.shape[0]

    src = edge_index[0].astype(jnp.int32)
    dst = edge_index[1].astype(jnp.int32)

    sc_msg = _make_sc_msg(n, d, e)

    wih_t = gru_w_ih.T
    whh_t = gru_w_hh.T
    bih = gru_b_ih.reshape(1, 3 * d)
    bhh = gru_b_hh.reshape(1, 3 * d)

    h = x
    m = _matmul(h, ggnn_w[0])
    for i in range(_T):
        agg2 = sc_msg(m, src, dst, edge_attr)
        a0 = agg2[0:n]
        a1 = agg2[n:2 * n]
        wnext = ggnn_w[i + 1] if i + 1 < _T else ggnn_w[0]
        h, m = _gru(a0, a1, h, wih_t, whh_t, bih, bhh, wnext)

    w1k = [conv1_w[:, :, t].T for t in range(3)]
    y1 = _h1y(h, w1k[0], w1k[1], w1k[2], conv1_b.reshape(1, d),
              bn_g.reshape(1, d), bn_b.reshape(1, d))
    lp = (y1.shape[0]) // 2
    y2 = _h2(y1[:2 * lp].reshape(lp, 2, d), conv2_w[:, :, 0].T,
             conv2_b.reshape(1, d), bn_g.reshape(1, d),
             bn_b.reshape(1, d))
    lq = y2.shape[0] // 2
    y3 = _h3(y2[:2 * lq].reshape(lq, 2, d), mlpy_w.T,
             mlpy_b.reshape(1, 1))

    whk = [conv1c_w[:, 0:d, t].T for t in range(3)]
    wxk = [conv1c_w[:, d:2 * d, t].T for t in range(3)]
    z1 = _h1z(h, x, whk, wxk, conv1c_b.reshape(1, c),
              bnc_g.reshape(1, c), bnc_b.reshape(1, c))
    z2 = _h2(z1[:2 * lp].reshape(lp, 2, c), conv2c_w[:, :, 0].T,
             conv2c_b.reshape(1, c), bnc_g.reshape(1, c),
             bnc_b.reshape(1, c))
    z3 = _h3(z2[:2 * lq].reshape(lq, 2, c), mlpz_w.T,
             mlpz_b.reshape(1, 1))

    out = _h4(y3, z3)
    return out.reshape((1,))


# speedup vs baseline: 9.1485x; 2.6940x over previous
import functools

import jax
import jax.numpy as jnp
from jax import lax
from jax.experimental import pallas as pl
from jax.experimental.pallas import tpu as pltpu
from jax.experimental.pallas import tpu_sc as plsc

_T = 6
_NC = 2
_NS = 16


@functools.cache
def _make_sc_msg(n, d, e):
    nw = _NC * _NS
    epw = e // nw
    k = 80
    assert epw % k == 0
    nchunks = epw // k
    nbuf = 4
    nmain = (nchunks - 1) // nbuf * nbuf
    assert nmain == nchunks - 1
    nouter = nmain // nbuf
    rpw = 624
    zr = 24
    nz = rpw // zr
    tail = n - _NS * rpw
    assert 0 <= tail <= zr and tail % 8 == 0

    mesh = plsc.VectorSubcoreMesh(
        core_axis_name="c", subcore_axis_name="s", num_cores=_NC)

    scratch = ([
        pltpu.VMEM((nbuf, k), jnp.int32),
        pltpu.VMEM((nbuf, k), jnp.int32),
        pltpu.VMEM((nbuf, k), jnp.float32),
    ] + [pltpu.VMEM((k, d), jnp.float32) for _ in range(nbuf)] + [
        pltpu.VMEM((zr, d), jnp.float32),
        pltpu.VMEM_SHARED((n, d), jnp.float32),
    ] + [pltpu.SemaphoreType.DMA] * (nbuf + 1))

    @functools.partial(
        pl.kernel,
        out_type=jax.ShapeDtypeStruct((_NC * n, d), jnp.float32),
        mesh=mesh,
        scratch_types=scratch,
    )
    def sc_msg(m_hbm, src_hbm, dst_hbm, ea_hbm, out_hbm, *scr):
        src_v, dst_v, ea_v = scr[0], scr[1], scr[2]
        rows = list(scr[3:3 + nbuf])
        zb_v = scr[3 + nbuf]
        agg_sh = scr[4 + nbuf]
        gsem = list(scr[5 + nbuf:5 + 2 * nbuf])
        isem = scr[5 + 2 * nbuf]

        c = lax.axis_index("c")
        s = lax.axis_index("s")
        ebase = (c * _NS + s) * epw

        def _se_start(ci, slot):
            off = ebase + ci * k
            pltpu.async_copy(src_hbm.at[pl.ds(off, k)], src_v.at[slot], isem)
            pltpu.async_copy(ea_hbm.at[pl.ds(off, k)], ea_v.at[slot], isem)

        def _d_start(ci, slot):
            off = ebase + ci * k
            pltpu.async_copy(dst_hbm.at[pl.ds(off, k)], dst_v.at[slot], isem)

        def _se_wait(ci, slot):
            off = ebase + ci * k
            pltpu.make_async_copy(src_hbm.at[pl.ds(off, k)],
                                  src_v.at[slot], isem).wait()
            pltpu.make_async_copy(ea_hbm.at[pl.ds(off, k)],
                                  ea_v.at[slot], isem).wait()

        def _d_wait(ci, slot):
            off = ebase + ci * k
            pltpu.make_async_copy(dst_hbm.at[pl.ds(off, k)],
                                  dst_v.at[slot], isem).wait()

        for b in range(nbuf - 1):
            _se_start(b, b)
            _d_start(b, b)

        zeros = jnp.zeros((16,), jnp.float32)

        def _zb(i, carry):
            for j in range(d // 16):
                zb_v[i, pl.ds(j * 16, 16)] = zeros
            return carry
        lax.fori_loop(0, zr, _zb, 0)
        for t in range(nz):
            pltpu.sync_copy(zb_v, agg_sh.at[pl.ds(s * rpw + t * zr, zr)])

        @pl.when(s == _NS - 1)
        def _zero_tail():
            pltpu.sync_copy(zb_v.at[pl.ds(0, tail)],
                            agg_sh.at[pl.ds(_NS * rpw, tail)])

        for b in range(nbuf - 1):
            _se_wait(b, b)

        for b in range(nbuf - 1):
            pltpu.async_copy(m_hbm.at[src_v.at[b]], rows[b], gsem[b])
        plsc.subcore_barrier()

        def _scale(b, ci_slot):
            def _group(g, cc):
                ea16 = ea_v[ci_slot, pl.ds(g * 16, 16)]
                for t in range(16):
                    scale = ea16.at[jnp.full((16,), t, jnp.int32)].get(
                        mode='promise_in_bounds')
                    for j in range(d // 16):
                        rows[b][g * 16 + t, pl.ds(j * 16, 16)] = (
                            rows[b][g * 16 + t, pl.ds(j * 16, 16)] * scale)
                return cc
            lax.fori_loop(0, k // 16, _group, 0)

        def _step(ci, b, refill):
            nb = (b + nbuf - 1) % nbuf
            nci = ci + nbuf - 1
            pltpu.make_async_copy(
                m_hbm.at[src_v.at[b]], rows[b], gsem[b]).wait()
            if refill:
                _se_start(nci, nb)
            _scale(b, b)
            _d_wait(ci, b)
            pltpu.sync_copy(rows[b], agg_sh.at[dst_v.at[b]], add=True)
            if refill:
                _se_wait(nci, nb)
                pltpu.async_copy(
                    m_hbm.at[src_v.at[nb]], rows[nb], gsem[nb])
                _d_start(nci, nb)

        for ci in range(nbuf):
            _step(ci, ci, refill=True)

        def _outer(i, carry):
            for b in range(nbuf):
                _step(i * nbuf + b, b, refill=True)
            return carry
        lax.fori_loop(1, nouter - 1, _outer, 0)

        for ci in range(nmain - nbuf, nchunks - 1):
            _step(ci, ci % nbuf, refill=ci + nbuf - 1 < nchunks)

        pltpu.make_async_copy(m_hbm.at[src_v.at[0]], rows[0], gsem[0]).wait()
        _scale(0, 0)
        _d_wait(nchunks - 1, 0)
        pltpu.sync_copy(rows[0], agg_sh.at[dst_v.at[0]], add=True)

        plsc.subcore_barrier()
        pltpu.sync_copy(agg_sh.at[pl.ds(s * rpw, rpw)],
                        out_hbm.at[pl.ds(c * n + s * rpw, rpw)])

        @pl.when(s == _NS - 1)
        def _write_tail():
            r0 = _NS * rpw
            pltpu.sync_copy(agg_sh.at[pl.ds(r0, tail)],
                            out_hbm.at[pl.ds(c * n + r0, tail)])

    return sc_msg


def _mm_body(h_ref, w_ref, o_ref):
    o_ref[...] = jnp.dot(h_ref[...], w_ref[...],
                         preferred_element_type=jnp.float32)


def _matmul(h, w):
    n, d = h.shape
    dout = w.shape[1]
    blk = 2000
    grid = n // blk
    return pl.pallas_call(
        _mm_body,
        grid=(grid,),
        in_specs=[
            pl.BlockSpec((blk, d), lambda i: (i, 0)),
            pl.BlockSpec((d, dout), lambda i: (0, 0)),
        ],
        out_specs=pl.BlockSpec((blk, dout), lambda i: (i, 0)),
        out_shape=jax.ShapeDtypeStruct((n, dout), jnp.float32),
    )(h, w)


def _gru_body(a0_ref, a1_ref, h_ref, wih_ref, whh_ref, bih_ref, bhh_ref,
              wnext_ref, hn_ref, mn_ref):
    d = h_ref.shape[1]
    a = a0_ref[...] + a1_ref[...]
    h = h_ref[...]
    gi = jnp.dot(a, wih_ref[...], preferred_element_type=jnp.float32) \
        + bih_ref[...]
    gh = jnp.dot(h, whh_ref[...], preferred_element_type=jnp.float32) \
        + bhh_ref[...]
    r = jax.nn.sigmoid(gi[:, 0:d] + gh[:, 0:d])
    z = jax.nn.sigmoid(gi[:, d:2 * d] + gh[:, d:2 * d])
    nn = jnp.tanh(gi[:, 2 * d:3 * d] + r * gh[:, 2 * d:3 * d])
    hn = (1.0 - z) * nn + z * h
    hn_ref[...] = hn
    mn_ref[...] = jnp.dot(hn, wnext_ref[...],
                          preferred_element_type=jnp.float32)


def _gru(a0, a1, h, wih_t, whh_t, bih, bhh, wnext):
    n, d = h.shape
    blk = 2000
    grid = n // blk
    return pl.pallas_call(
        _gru_body,
        grid=(grid,),
        in_specs=[
            pl.BlockSpec((blk, d), lambda i: (i, 0)),
            pl.BlockSpec((blk, d), lambda i: (i, 0)),
            pl.BlockSpec((blk, d), lambda i: (i, 0)),
            pl.BlockSpec((d, 3 * d), lambda i: (0, 0)),
            pl.BlockSpec((d, 3 * d), lambda i: (0, 0)),
            pl.BlockSpec((1, 3 * d), lambda i: (0, 0)),
            pl.BlockSpec((1, 3 * d), lambda i: (0, 0)),
            pl.BlockSpec((d, d), lambda i: (0, 0)),
        ],
        out_specs=[
            pl.BlockSpec((blk, d), lambda i: (i, 0)),
            pl.BlockSpec((blk, d), lambda i: (i, 0)),
        ],
        out_shape=[
            jax.ShapeDtypeStruct((n, d), jnp.float32),
            jax.ShapeDtypeStruct((n, d), jnp.float32),
        ],
    )(a0, a1, h, wih_t, whh_t, bih, bhh, wnext)


def _h1y_body(x_ref, w0_ref, w1_ref, w2_ref, b_ref, g_ref, be_ref, y_ref):
    lc = y_ref.shape[0]
    c1 = (jnp.dot(x_ref[pl.ds(0, lc), :], w0_ref[...],
                  preferred_element_type=jnp.float32)
          + jnp.dot(x_ref[pl.ds(1, lc), :], w1_ref[...],
                    preferred_element_type=jnp.float32)
          + jnp.dot(x_ref[pl.ds(2, lc), :], w2_ref[...],
                    preferred_element_type=jnp.float32)
          + b_ref[...])
    mu = jnp.mean(c1, axis=0, keepdims=True)
    var = jnp.mean(c1 * c1, axis=0, keepdims=True) - mu * mu
    y = (c1 - mu) * lax.rsqrt(var + 1e-5) * g_ref[...] + be_ref[...]
    y_ref[...] = jnp.maximum(y, 0.0)


def _h1y(x, w0, w1, w2, b, g, be):
    l, d = x.shape
    ch = w0.shape[1]
    lc = l - 2
    return pl.pallas_call(
        _h1y_body,
        out_shape=jax.ShapeDtypeStruct((lc, ch), jnp.float32),
    )(x, w0, w1, w2, b, g, be)


def _h1z_body(h_ref, x_ref, wh0_ref, wh1_ref, wh2_ref,
              wx0_ref, wx1_ref, wx2_ref, b_ref, g_ref, be_ref, y_ref):
    lc = y_ref.shape[0]
    c1 = b_ref[...]
    for (src_ref, wrefs) in ((h_ref, (wh0_ref, wh1_ref, wh2_ref)),
                             (x_ref, (wx0_ref, wx1_ref, wx2_ref))):
        for t in range(3):
            c1 = c1 + jnp.dot(src_ref[pl.ds(t, lc), :], wrefs[t][...],
                              preferred_element_type=jnp.float32)
    mu = jnp.mean(c1, axis=0, keepdims=True)
    var = jnp.mean(c1 * c1, axis=0, keepdims=True) - mu * mu
    y = (c1 - mu) * lax.rsqrt(var + 1e-5) * g_ref[...] + be_ref[...]
    y_ref[...] = jnp.maximum(y, 0.0)


def _h1z(h, x, wh, wx, b, g, be):
    l, d = h.shape
    ch = wh[0].shape[1]
    lc = l - 2
    return pl.pallas_call(
        _h1z_body,
        out_shape=jax.ShapeDtypeStruct((lc, ch), jnp.float32),
    )(h, x, wh[0], wh[1], wh[2], wx[0], wx[1], wx[2], b, g, be)


def _h2_body(p_ref, w2_ref, b2_ref, g_ref, be_ref, y_ref):
    lo = y_ref.shape[0]
    v = p_ref[...]
    pm = jnp.maximum(v[:, 0, :], v[:, 1, :])
    ev = v[:, 0, :]
    out1 = jnp.maximum(pm[0:lo, :], ev[1:lo + 1, :])
    c2 = jnp.dot(out1, w2_ref[...],
                 preferred_element_type=jnp.float32) + b2_ref[...]
    mu = jnp.mean(c2, axis=0, keepdims=True)
    var = jnp.mean(c2 * c2, axis=0, keepdims=True) - mu * mu
    y = (c2 - mu) * lax.rsqrt(var + 1e-5) * g_ref[...] + be_ref[...]
    y_ref[...] = jnp.maximum(y, 0.0)


def _h2(p, w2, b2, g, be):
    lp, _, ch = p.shape
    lo = lp - 1
    return pl.pallas_call(
        _h2_body,
        out_shape=jax.ShapeDtypeStruct((lo, ch), jnp.float32),
    )(p, w2, b2, g, be)


def _h3_body(p_ref, mw_ref, mb_ref, o_ref):
    v = p_ref[...]
    pm = jnp.maximum(v[:, 0, :], v[:, 1, :])
    o_ref[...] = jnp.dot(pm, mw_ref[...],
                         preferred_element_type=jnp.float32) + mb_ref[...]


def _h3(p, mw, mb):
    lp = p.shape[0]
    return pl.pallas_call(
        _h3_body,
        out_shape=jax.ShapeDtypeStruct((lp, 1), jnp.float32),
    )(p, mw, mb)


def _h4_body(y_ref, z_ref, o_ref):
    prod = y_ref[...] * z_ref[...]
    m = jnp.sum(prod, axis=0, keepdims=True) / prod.shape[0]
    o_ref[...] = jax.nn.sigmoid(m)


def _h4(y, z):
    return pl.pallas_call(
        _h4_body,
        out_shape=jax.ShapeDtypeStruct((1, 1), jnp.float32),
    )(y, z)


def kernel(x, edge_index, edge_attr, ggnn_w, gru_w_ih, gru_w_hh, gru_b_ih,
           gru_b_hh, conv1_w, conv1_b, conv2_w, conv2_b, conv1c_w, conv1c_b,
           conv2c_w, conv2c_b, bn_g, bn_b, bnc_g, bnc_b, mlpy_w, mlpy_b,
           mlpz_w, mlpz_b):
    n, d = x.shape
    e = edge_attr.shape[0]
    c = conv1c_w.shape[0]

    src = edge_index[0].astype(jnp.int32)
    dst = edge_index[1].astype(jnp.int32)

    sc_msg = _make_sc_msg(n, d, e)

    wih_t = gru_w_ih.T
    whh_t = gru_w_hh.T
    bih = gru_b_ih.reshape(1, 3 * d)
    bhh = gru_b_hh.reshape(1, 3 * d)

    h = x
    m = _matmul(h, ggnn_w[0])
    for i in range(_T):
        agg2 = sc_msg(m, src, dst, edge_attr)
        a0 = agg2[0:n]
        a1 = agg2[n:2 * n]
        wnext = ggnn_w[i + 1] if i + 1 < _T else ggnn_w[0]
        h, m = _gru(a0, a1, h, wih_t, whh_t, bih, bhh, wnext)

    w1k = [conv1_w[:, :, t].T for t in range(3)]
    y1 = _h1y(h, w1k[0], w1k[1], w1k[2], conv1_b.reshape(1, d),
              bn_g.reshape(1, d), bn_b.reshape(1, d))
    lp = (y1.shape[0]) // 2
    y2 = _h2(y1[:2 * lp].reshape(lp, 2, d), conv2_w[:, :, 0].T,
             conv2_b.reshape(1, d), bn_g.reshape(1, d),
             bn_b.reshape(1, d))
    lq = y2.shape[0] // 2
    y3 = _h3(y2[:2 * lq].reshape(lq, 2, d), mlpy_w.T,
             mlpy_b.reshape(1, 1))

    whk = [conv1c_w[:, 0:d, t].T for t in range(3)]
    wxk = [conv1c_w[:, d:2 * d, t].T for t in range(3)]
    z1 = _h1z(h, x, whk, wxk, conv1c_b.reshape(1, c),
              bnc_g.reshape(1, c), bnc_b.reshape(1, c))
    z2 = _h2(z1[:2 * lp].reshape(lp, 2, c), conv2c_w[:, :, 0].T,
             conv2c_b.reshape(1, c), bnc_g.reshape(1, c),
             bnc_b.reshape(1, c))
    z3 = _h3(z2[:2 * lq].reshape(lq, 2, c), mlpz_w.T,
             mlpz_b.reshape(1, 1))

    out = _h4(y3, z3)
    return out.reshape((1,))


# speedup vs baseline: 9.6358x; 1.0533x over previous
import functools

import jax
import jax.numpy as jnp
from jax import lax
from jax.experimental import pallas as pl
from jax.experimental.pallas import tpu as pltpu
from jax.experimental.pallas import tpu_sc as plsc

_T = 6
_NC = 2
_NS = 16


@functools.cache
def _make_sc_msg(n, d, e):
    nw = _NC * _NS
    epw = e // nw
    k = 80
    assert epw % k == 0
    nchunks = epw // k
    nbuf = 4
    nmain = (nchunks - 1) // nbuf * nbuf
    assert nmain == nchunks - 1
    nouter = nmain // nbuf
    rpw = 624
    zr = 24
    nz = rpw // zr
    tail = n - _NS * rpw
    assert 0 <= tail <= zr and tail % 8 == 0

    mesh = plsc.VectorSubcoreMesh(
        core_axis_name="c", subcore_axis_name="s", num_cores=_NC)

    scratch = ([
        pltpu.VMEM((nbuf, k), jnp.int32),
        pltpu.VMEM((nbuf, k), jnp.int32),
        pltpu.VMEM((nbuf, k), jnp.float32),
    ] + [pltpu.VMEM((k, d), jnp.float32) for _ in range(nbuf)] + [
        pltpu.VMEM((zr, d), jnp.float32),
        pltpu.VMEM_SHARED((n, d), jnp.float32),
    ] + [pltpu.SemaphoreType.DMA] * (nbuf + 1))

    @functools.partial(
        pl.kernel,
        out_type=jax.ShapeDtypeStruct((_NC * n, d), jnp.float32),
        mesh=mesh,
        scratch_types=scratch,
    )
    def sc_msg(m_hbm, src_hbm, dst_hbm, ea_hbm, out_hbm, *scr):
        src_v, dst_v, ea_v = scr[0], scr[1], scr[2]
        rows = list(scr[3:3 + nbuf])
        zb_v = scr[3 + nbuf]
        agg_sh = scr[4 + nbuf]
        gsem = list(scr[5 + nbuf:5 + 2 * nbuf])
        isem = scr[5 + 2 * nbuf]

        c = lax.axis_index("c")
        s = lax.axis_index("s")
        ebase = (c * _NS + s) * epw

        def _se_start(ci, slot):
            off = ebase + ci * k
            pltpu.async_copy(src_hbm.at[pl.ds(off, k)], src_v.at[slot], isem)
            pltpu.async_copy(ea_hbm.at[pl.ds(off, k)], ea_v.at[slot], isem)

        def _d_start(ci, slot):
            off = ebase + ci * k
            pltpu.async_copy(dst_hbm.at[pl.ds(off, k)], dst_v.at[slot], isem)

        def _se_wait(ci, slot):
            off = ebase + ci * k
            pltpu.make_async_copy(src_hbm.at[pl.ds(off, k)],
                                  src_v.at[slot], isem).wait()
            pltpu.make_async_copy(ea_hbm.at[pl.ds(off, k)],
                                  ea_v.at[slot], isem).wait()

        def _d_wait(ci, slot):
            off = ebase + ci * k
            pltpu.make_async_copy(dst_hbm.at[pl.ds(off, k)],
                                  dst_v.at[slot], isem).wait()

        for b in range(nbuf - 1):
            _se_start(b, b)
            _d_start(b, b)

        zeros = jnp.zeros((16,), jnp.float32)

        def _zb(i, carry):
            for j in range(d // 16):
                zb_v[i, pl.ds(j * 16, 16)] = zeros
            return carry
        lax.fori_loop(0, zr, _zb, 0)
        for t in range(nz):
            pltpu.sync_copy(zb_v, agg_sh.at[pl.ds(s * rpw + t * zr, zr)])

        @pl.when(s == _NS - 1)
        def _zero_tail():
            pltpu.sync_copy(zb_v.at[pl.ds(0, tail)],
                            agg_sh.at[pl.ds(_NS * rpw, tail)])

        for b in range(nbuf - 1):
            _se_wait(b, b)

        for b in range(nbuf - 1):
            pltpu.async_copy(m_hbm.at[src_v.at[b]], rows[b], gsem[b])
        plsc.subcore_barrier()

        def _scale(b, ci_slot):
            def _group(g, cc):
                ea16 = ea_v[ci_slot, pl.ds(g * 16, 16)]
                for t in range(16):
                    scale = ea16.at[jnp.full((16,), t, jnp.int32)].get(
                        mode='promise_in_bounds')
                    for j in range(d // 16):
                        rows[b][g * 16 + t, pl.ds(j * 16, 16)] = (
                            rows[b][g * 16 + t, pl.ds(j * 16, 16)] * scale)
                return cc
            lax.fori_loop(0, k // 16, _group, 0)

        def _step(ci, b, refill):
            nb = (b + nbuf - 1) % nbuf
            nci = ci + nbuf - 1
            pltpu.make_async_copy(
                m_hbm.at[src_v.at[b]], rows[b], gsem[b]).wait()
            if refill:
                _se_start(nci, nb)
            _scale(b, b)
            _d_wait(ci, b)
            pltpu.sync_copy(rows[b], agg_sh.at[dst_v.at[b]], add=True)
            if refill:
                _se_wait(nci, nb)
                pltpu.async_copy(
                    m_hbm.at[src_v.at[nb]], rows[nb], gsem[nb])
                _d_start(nci, nb)

        for ci in range(nbuf):
            _step(ci, ci, refill=True)

        def _outer(i, carry):
            for b in range(nbuf):
                _step(i * nbuf + b, b, refill=True)
            return carry
        lax.fori_loop(1, nouter - 1, _outer, 0)

        for ci in range(nmain - nbuf, nchunks - 1):
            _step(ci, ci % nbuf, refill=ci + nbuf - 1 < nchunks)

        pltpu.make_async_copy(m_hbm.at[src_v.at[0]], rows[0], gsem[0]).wait()
        _scale(0, 0)
        _d_wait(nchunks - 1, 0)
        pltpu.sync_copy(rows[0], agg_sh.at[dst_v.at[0]], add=True)

        plsc.subcore_barrier()
        pltpu.sync_copy(agg_sh.at[pl.ds(s * rpw, rpw)],
                        out_hbm.at[pl.ds(c * n + s * rpw, rpw)])

        @pl.when(s == _NS - 1)
        def _write_tail():
            r0 = _NS * rpw
            pltpu.sync_copy(agg_sh.at[pl.ds(r0, tail)],
                            out_hbm.at[pl.ds(c * n + r0, tail)])

    return sc_msg


def _mm_body(h_ref, w_ref, o_ref):
    o_ref[...] = jnp.dot(h_ref[...], w_ref[...],
                         preferred_element_type=jnp.float32)


def _matmul(h, w):
    n, d = h.shape
    dout = w.shape[1]
    blk = 2000
    grid = n // blk
    return pl.pallas_call(
        _mm_body,
        grid=(grid,),
        in_specs=[
            pl.BlockSpec((blk, d), lambda i: (i, 0)),
            pl.BlockSpec((d, dout), lambda i: (0, 0)),
        ],
        out_specs=pl.BlockSpec((blk, dout), lambda i: (i, 0)),
        out_shape=jax.ShapeDtypeStruct((n, dout), jnp.float32),
    )(h, w)


def _gru_body(a0_ref, a1_ref, h_ref, wih_ref, whh_ref, bih_ref, bhh_ref,
              wnext_ref, hn_ref, mn_ref):
    d = h_ref.shape[1]
    a = a0_ref[...] + a1_ref[...]
    h = h_ref[...]
    gi = jnp.dot(a, wih_ref[...], preferred_element_type=jnp.float32) \
        + bih_ref[...]
    gh = jnp.dot(h, whh_ref[...], preferred_element_type=jnp.float32) \
        + bhh_ref[...]
    r = jax.nn.sigmoid(gi[:, 0:d] + gh[:, 0:d])
    z = jax.nn.sigmoid(gi[:, d:2 * d] + gh[:, d:2 * d])
    nn = jnp.tanh(gi[:, 2 * d:3 * d] + r * gh[:, 2 * d:3 * d])
    hn = (1.0 - z) * nn + z * h
    hn_ref[...] = hn
    mn_ref[...] = jnp.dot(hn, wnext_ref[...],
                          preferred_element_type=jnp.float32)


def _gru(a0, a1, h, wih_t, whh_t, bih, bhh, wnext):
    n, d = h.shape
    blk = 2000
    grid = n // blk
    return pl.pallas_call(
        _gru_body,
        grid=(grid,),
        in_specs=[
            pl.BlockSpec((blk, d), lambda i: (i, 0)),
            pl.BlockSpec((blk, d), lambda i: (i, 0)),
            pl.BlockSpec((blk, d), lambda i: (i, 0)),
            pl.BlockSpec((d, 3 * d), lambda i: (0, 0)),
            pl.BlockSpec((d, 3 * d), lambda i: (0, 0)),
            pl.BlockSpec((1, 3 * d), lambda i: (0, 0)),
            pl.BlockSpec((1, 3 * d), lambda i: (0, 0)),
            pl.BlockSpec((d, d), lambda i: (0, 0)),
        ],
        out_specs=[
            pl.BlockSpec((blk, d), lambda i: (i, 0)),
            pl.BlockSpec((blk, d), lambda i: (i, 0)),
        ],
        out_shape=[
            jax.ShapeDtypeStruct((n, d), jnp.float32),
            jax.ShapeDtypeStruct((n, d), jnp.float32),
        ],
    )(a0, a1, h, wih_t, whh_t, bih, bhh, wnext)


def _bn_relu(c, g, be):
    mu = jnp.mean(c, axis=0, keepdims=True)
    var = jnp.mean(c * c, axis=0, keepdims=True) - mu * mu
    return jnp.maximum((c - mu) * lax.rsqrt(var + 1e-5) * g + be, 0.0)


def _head_tail(y, w2, b2, g, be, mw, mb):
    lc, ch = y.shape
    lp = lc // 2
    p = y.reshape(lp, 2, ch)
    pm = jnp.maximum(p[:, 0, :], p[:, 1, :])
    ev = p[:, 0, :]
    out1 = jnp.maximum(pm[0:lp - 1, :], ev[1:lp, :])
    y2 = _bn_relu(jnp.dot(out1, w2, preferred_element_type=jnp.float32)
                  + b2, g, be)
    lq = (lp - 1) // 2
    q = y2.reshape(lq, 2, ch)
    y2p = jnp.maximum(q[:, 0, :], q[:, 1, :])
    return jnp.dot(y2p, mw, preferred_element_type=jnp.float32) + mb


def _hy_body(x_ref, w0_ref, w1_ref, w2_ref, b_ref, g_ref, be_ref,
             cw2_ref, cb2_ref, mw_ref, mb_ref, o_ref):
    lc = x_ref.shape[0] - 2
    c1 = (jnp.dot(x_ref[pl.ds(0, lc), :], w0_ref[...],
                  preferred_element_type=jnp.float32)
          + jnp.dot(x_ref[pl.ds(1, lc), :], w1_ref[...],
                    preferred_element_type=jnp.float32)
          + jnp.dot(x_ref[pl.ds(2, lc), :], w2_ref[...],
                    preferred_element_type=jnp.float32)
          + b_ref[...])
    y = _bn_relu(c1, g_ref[...], be_ref[...])
    o_ref[...] = _head_tail(y, cw2_ref[...], cb2_ref[...], g_ref[...],
                            be_ref[...], mw_ref[...], mb_ref[...])


def _hy(x, w0, w1, w2, b, g, be, cw2, cb2, mw, mb):
    l = x.shape[0]
    lq = (l - 2) // 2 // 2
    return pl.pallas_call(
        _hy_body,
        out_shape=jax.ShapeDtypeStruct((lq, 1), jnp.float32),
    )(x, w0, w1, w2, b, g, be, cw2, cb2, mw, mb)


def _hz_body(h_ref, x_ref, wh0_ref, wh1_ref, wh2_ref,
             wx0_ref, wx1_ref, wx2_ref, b_ref, g_ref, be_ref,
             cw2_ref, cb2_ref, mw_ref, mb_ref, o_ref):
    lc = h_ref.shape[0] - 2
    c1 = b_ref[...]
    for (src_ref, wrefs) in ((h_ref, (wh0_ref, wh1_ref, wh2_ref)),
                             (x_ref, (wx0_ref, wx1_ref, wx2_ref))):
        for t in range(3):
            c1 = c1 + jnp.dot(src_ref[pl.ds(t, lc), :], wrefs[t][...],
                              preferred_element_type=jnp.float32)
    y = _bn_relu(c1, g_ref[...], be_ref[...])
    o_ref[...] = _head_tail(y, cw2_ref[...], cb2_ref[...], g_ref[...],
                            be_ref[...], mw_ref[...], mb_ref[...])


def _hz(h, x, wh, wx, b, g, be, cw2, cb2, mw, mb):
    l = h.shape[0]
    lq = (l - 2) // 2 // 2
    return pl.pallas_call(
        _hz_body,
        out_shape=jax.ShapeDtypeStruct((lq, 1), jnp.float32),
    )(h, x, wh[0], wh[1], wh[2], wx[0], wx[1], wx[2], b, g, be,
      cw2, cb2, mw, mb)


def _h4_body(y_ref, z_ref, o_ref):
    prod = y_ref[...] * z_ref[...]
    m = jnp.sum(prod, axis=0, keepdims=True) / prod.shape[0]
    o_ref[...] = jax.nn.sigmoid(m)


def _h4(y, z):
    return pl.pallas_call(
        _h4_body,
        out_shape=jax.ShapeDtypeStruct((1, 1), jnp.float32),
    )(y, z)


def kernel(x, edge_index, edge_attr, ggnn_w, gru_w_ih, gru_w_hh, gru_b_ih,
           gru_b_hh, conv1_w, conv1_b, conv2_w, conv2_b, conv1c_w, conv1c_b,
           conv2c_w, conv2c_b, bn_g, bn_b, bnc_g, bnc_b, mlpy_w, mlpy_b,
           mlpz_w, mlpz_b):
    n, d = x.shape
    e = edge_attr.shape[0]
    c = conv1c_w.shape[0]

    src = edge_index[0].astype(jnp.int32)
    dst = edge_index[1].astype(jnp.int32)

    sc_msg = _make_sc_msg(n, d, e)

    wih_t = gru_w_ih.T
    whh_t = gru_w_hh.T
    bih = gru_b_ih.reshape(1, 3 * d)
    bhh = gru_b_hh.reshape(1, 3 * d)

    h = x
    m = _matmul(h, ggnn_w[0])
    for i in range(_T):
        agg2 = sc_msg(m, src, dst, edge_attr)
        a0 = agg2[0:n]
        a1 = agg2[n:2 * n]
        wnext = ggnn_w[i + 1] if i + 1 < _T else ggnn_w[0]
        h, m = _gru(a0, a1, h, wih_t, whh_t, bih, bhh, wnext)

    w1k = [conv1_w[:, :, t].T for t in range(3)]
    y3 = _hy(h, w1k[0], w1k[1], w1k[2], conv1_b.reshape(1, d),
             bn_g.reshape(1, d), bn_b.reshape(1, d), conv2_w[:, :, 0].T,
             conv2_b.reshape(1, d), mlpy_w.T,
             mlpy_b.reshape(1, 1))

    whk = [conv1c_w[:, 0:d, t].T for t in range(3)]
    wxk = [conv1c_w[:, d:2 * d, t].T for t in range(3)]
    z3 = _hz(h, x, whk, wxk, conv1c_b.reshape(1, c),
             bnc_g.reshape(1, c), bnc_b.reshape(1, c), conv2c_w[:, :, 0].T,
             conv2c_b.reshape(1, c), mlpz_w.T,
             mlpz_b.reshape(1, 1))

    out = _h4(y3, z3)
    return out.reshape((1,))


# speedup vs baseline: 10.0666x; 1.0447x over previous
import functools

import jax
import jax.numpy as jnp
from jax import lax
from jax.experimental import pallas as pl
from jax.experimental.pallas import tpu as pltpu
from jax.experimental.pallas import tpu_sc as plsc

_T = 6
_NC = 2
_NS = 16


@functools.cache
def _make_sc_msg(n, d, e):
    nw = _NC * _NS
    epw = e // nw
    k = 80
    assert epw % k == 0
    nchunks = epw // k
    nbuf = 4
    nmain = (nchunks - 1) // nbuf * nbuf
    assert nmain == nchunks - 1
    nouter = nmain // nbuf
    rpw = 624
    zr = 24
    nz = rpw // zr
    tail = n - _NS * rpw
    assert 0 <= tail <= zr and tail % 8 == 0

    mesh = plsc.VectorSubcoreMesh(
        core_axis_name="c", subcore_axis_name="s", num_cores=_NC)

    scratch = ([
        pltpu.VMEM((nbuf, k), jnp.int32),
        pltpu.VMEM((nbuf, k), jnp.int32),
        pltpu.VMEM((nbuf, k), jnp.float32),
    ] + [pltpu.VMEM((k, d), jnp.float32) for _ in range(nbuf)] + [
        pltpu.VMEM((zr, d), jnp.float32),
        pltpu.VMEM_SHARED((n, d), jnp.float32),
    ] + [pltpu.SemaphoreType.DMA] * (nbuf + 1))

    @functools.partial(
        pl.kernel,
        out_type=jax.ShapeDtypeStruct((_NC * n, d), jnp.float32),
        mesh=mesh,
        scratch_types=scratch,
    )
    def sc_msg(m_hbm, src_hbm, dst_hbm, ea_hbm, out_hbm, *scr):
        src_v, dst_v, ea_v = scr[0], scr[1], scr[2]
        rows = list(scr[3:3 + nbuf])
        zb_v = scr[3 + nbuf]
        agg_sh = scr[4 + nbuf]
        gsem = list(scr[5 + nbuf:5 + 2 * nbuf])
        isem = scr[5 + 2 * nbuf]

        c = lax.axis_index("c")
        s = lax.axis_index("s")
        ebase = (c * _NS + s) * epw

        def _se_start(ci, slot):
            off = ebase + ci * k
            pltpu.async_copy(src_hbm.at[pl.ds(off, k)], src_v.at[slot], isem)
            pltpu.async_copy(ea_hbm.at[pl.ds(off, k)], ea_v.at[slot], isem)

        def _d_start(ci, slot):
            off = ebase + ci * k
            pltpu.async_copy(dst_hbm.at[pl.ds(off, k)], dst_v.at[slot], isem)

        def _se_wait(ci, slot):
            off = ebase + ci * k
            pltpu.make_async_copy(src_hbm.at[pl.ds(off, k)],
                                  src_v.at[slot], isem).wait()
            pltpu.make_async_copy(ea_hbm.at[pl.ds(off, k)],
                                  ea_v.at[slot], isem).wait()

        def _d_wait(ci, slot):
            off = ebase + ci * k
            pltpu.make_async_copy(dst_hbm.at[pl.ds(off, k)],
                                  dst_v.at[slot], isem).wait()

        for b in range(nbuf - 1):
            _se_start(b, b)
            _d_start(b, b)

        zeros = jnp.zeros((16,), jnp.float32)

        def _zb(i, carry):
            for j in range(d // 16):
                zb_v[i, pl.ds(j * 16, 16)] = zeros
            return carry
        lax.fori_loop(0, zr, _zb, 0)
        for t in range(nz):
            pltpu.sync_copy(zb_v, agg_sh.at[pl.ds(s * rpw + t * zr, zr)])

        @pl.when(s == _NS - 1)
        def _zero_tail():
            pltpu.sync_copy(zb_v.at[pl.ds(0, tail)],
                            agg_sh.at[pl.ds(_NS * rpw, tail)])

        for b in range(nbuf - 1):
            _se_wait(b, b)

        for b in range(nbuf - 1):
            pltpu.async_copy(m_hbm.at[src_v.at[b]], rows[b], gsem[b])
        plsc.subcore_barrier()

        def _scale(b, ci_slot):
            def _group(g, cc):
                ea16 = ea_v[ci_slot, pl.ds(g * 16, 16)]
                for t in range(16):
                    scale = ea16.at[jnp.full((16,), t, jnp.int32)].get(
                        mode='promise_in_bounds')
                    for j in range(d // 16):
                        rows[b][g * 16 + t, pl.ds(j * 16, 16)] = (
                            rows[b][g * 16 + t, pl.ds(j * 16, 16)] * scale)
                return cc
            lax.fori_loop(0, k // 16, _group, 0)

        def _step(ci, b, refill):
            nb = (b + nbuf - 1) % nbuf
            nci = ci + nbuf - 1
            pltpu.make_async_copy(
                m_hbm.at[src_v.at[b]], rows[b], gsem[b]).wait()
            if refill:
                _se_start(nci, nb)
            _scale(b, b)
            _d_wait(ci, b)
            pltpu.sync_copy(rows[b], agg_sh.at[dst_v.at[b]], add=True)
            if refill:
                _se_wait(nci, nb)
                pltpu.async_copy(
                    m_hbm.at[src_v.at[nb]], rows[nb], gsem[nb])
                _d_start(nci, nb)

        for ci in range(nbuf):
            _step(ci, ci, refill=True)

        def _outer(i, carry):
            for b in range(nbuf):
                _step(i * nbuf + b, b, refill=True)
            return carry
        lax.fori_loop(1, nouter - 1, _outer, 0)

        for ci in range(nmain - nbuf, nchunks - 1):
            _step(ci, ci % nbuf, refill=ci + nbuf - 1 < nchunks)

        pltpu.make_async_copy(m_hbm.at[src_v.at[0]], rows[0], gsem[0]).wait()
        _scale(0, 0)
        _d_wait(nchunks - 1, 0)
        pltpu.sync_copy(rows[0], agg_sh.at[dst_v.at[0]], add=True)

        plsc.subcore_barrier()
        pltpu.sync_copy(agg_sh.at[pl.ds(s * rpw, rpw)],
                        out_hbm.at[pl.ds(c * n + s * rpw, rpw)])

        @pl.when(s == _NS - 1)
        def _write_tail():
            r0 = _NS * rpw
            pltpu.sync_copy(agg_sh.at[pl.ds(r0, tail)],
                            out_hbm.at[pl.ds(c * n + r0, tail)])

    return sc_msg


def _mm_body(h_ref, w_ref, o_ref):
    o_ref[...] = jnp.dot(h_ref[...], w_ref[...],
                         preferred_element_type=jnp.float32)


def _matmul(h, w):
    n, d = h.shape
    dout = w.shape[1]
    blk = 2000
    grid = n // blk
    return pl.pallas_call(
        _mm_body,
        grid=(grid,),
        in_specs=[
            pl.BlockSpec((blk, d), lambda i: (i, 0)),
            pl.BlockSpec((d, dout), lambda i: (0, 0)),
        ],
        out_specs=pl.BlockSpec((blk, dout), lambda i: (i, 0)),
        out_shape=jax.ShapeDtypeStruct((n, dout), jnp.float32),
    )(h, w)


def _gru_body(a0_ref, a1_ref, h_ref, wih_ref, whh_ref, bih_ref, bhh_ref,
              wnext_ref, hn_ref, mn_ref):
    d = h_ref.shape[1]
    a = a0_ref[...] + a1_ref[...]
    h = h_ref[...]
    gi = jnp.dot(a, wih_ref[...], preferred_element_type=jnp.float32) \
        + bih_ref[...]
    gh = jnp.dot(h, whh_ref[...], preferred_element_type=jnp.float32) \
        + bhh_ref[...]
    r = jax.nn.sigmoid(gi[:, 0:d] + gh[:, 0:d])
    z = jax.nn.sigmoid(gi[:, d:2 * d] + gh[:, d:2 * d])
    nn = jnp.tanh(gi[:, 2 * d:3 * d] + r * gh[:, 2 * d:3 * d])
    hn = (1.0 - z) * nn + z * h
    hn_ref[...] = hn
    mn_ref[...] = jnp.dot(hn, wnext_ref[...],
                          preferred_element_type=jnp.float32)


def _gru(agg2, h, wih_t, whh_t, bih, bhh, wnext):
    n, d = h.shape
    blk = 2000
    grid = n // blk
    return pl.pallas_call(
        _gru_body,
        grid=(grid,),
        in_specs=[
            pl.BlockSpec((blk, d), lambda i: (i, 0)),
            pl.BlockSpec((blk, d), lambda i, g=grid: (i + g, 0)),
            pl.BlockSpec((blk, d), lambda i: (i, 0)),
            pl.BlockSpec((d, 3 * d), lambda i: (0, 0)),
            pl.BlockSpec((d, 3 * d), lambda i: (0, 0)),
            pl.BlockSpec((1, 3 * d), lambda i: (0, 0)),
            pl.BlockSpec((1, 3 * d), lambda i: (0, 0)),
            pl.BlockSpec((d, d), lambda i: (0, 0)),
        ],
        out_specs=[
            pl.BlockSpec((blk, d), lambda i: (i, 0)),
            pl.BlockSpec((blk, d), lambda i: (i, 0)),
        ],
        out_shape=[
            jax.ShapeDtypeStruct((n, d), jnp.float32),
            jax.ShapeDtypeStruct((n, d), jnp.float32),
        ],
    )(agg2, agg2, h, wih_t, whh_t, bih, bhh, wnext)


def _bn_relu(c, g, be):
    mu = jnp.mean(c, axis=0, keepdims=True)
    var = jnp.mean(c * c, axis=0, keepdims=True) - mu * mu
    return jnp.maximum((c - mu) * lax.rsqrt(var + 1e-5) * g + be, 0.0)


def _head_tail(y, w2, b2, g, be, mw, mb):
    lc, ch = y.shape
    lp = lc // 2
    p = y.reshape(lp, 2, ch)
    pm = jnp.maximum(p[:, 0, :], p[:, 1, :])
    ev = p[:, 0, :]
    out1 = jnp.maximum(pm[0:lp - 1, :], ev[1:lp, :])
    y2 = _bn_relu(jnp.dot(out1, w2, preferred_element_type=jnp.float32)
                  + b2, g, be)
    lq = (lp - 1) // 2
    q = y2.reshape(lq, 2, ch)
    y2p = jnp.maximum(q[:, 0, :], q[:, 1, :])
    return jnp.dot(y2p, mw, preferred_element_type=jnp.float32) + mb


def _hy_body(x_ref, w0_ref, w1_ref, w2_ref, b_ref, g_ref, be_ref,
             cw2_ref, cb2_ref, mw_ref, mb_ref, o_ref):
    lc = x_ref.shape[0] - 2
    c1 = (jnp.dot(x_ref[pl.ds(0, lc), :], w0_ref[...],
                  preferred_element_type=jnp.float32)
          + jnp.dot(x_ref[pl.ds(1, lc), :], w1_ref[...],
                    preferred_element_type=jnp.float32)
          + jnp.dot(x_ref[pl.ds(2, lc), :], w2_ref[...],
                    preferred_element_type=jnp.float32)
          + b_ref[...])
    y = _bn_relu(c1, g_ref[...], be_ref[...])
    o_ref[...] = _head_tail(y, cw2_ref[...], cb2_ref[...], g_ref[...],
                            be_ref[...], mw_ref[...], mb_ref[...])


def _hy(x, w0, w1, w2, b, g, be, cw2, cb2, mw, mb):
    l = x.shape[0]
    lq = (l - 2) // 2 // 2
    return pl.pallas_call(
        _hy_body,
        out_shape=jax.ShapeDtypeStruct((lq, 1), jnp.float32),
    )(x, w0, w1, w2, b, g, be, cw2, cb2, mw, mb)


def _hz_body(h_ref, x_ref, wh0_ref, wh1_ref, wh2_ref,
             wx0_ref, wx1_ref, wx2_ref, b_ref, g_ref, be_ref,
             cw2_ref, cb2_ref, mw_ref, mb_ref, o_ref):
    lc = h_ref.shape[0] - 2
    c1 = b_ref[...]
    for (src_ref, wrefs) in ((h_ref, (wh0_ref, wh1_ref, wh2_ref)),
                             (x_ref, (wx0_ref, wx1_ref, wx2_ref))):
        for t in range(3):
            c1 = c1 + jnp.dot(src_ref[pl.ds(t, lc), :], wrefs[t][...],
                              preferred_element_type=jnp.float32)
    y = _bn_relu(c1, g_ref[...], be_ref[...])
    o_ref[...] = _head_tail(y, cw2_ref[...], cb2_ref[...], g_ref[...],
                            be_ref[...], mw_ref[...], mb_ref[...])


def _hz(h, x, wh, wx, b, g, be, cw2, cb2, mw, mb):
    l = h.shape[0]
    lq = (l - 2) // 2 // 2
    return pl.pallas_call(
        _hz_body,
        out_shape=jax.ShapeDtypeStruct((lq, 1), jnp.float32),
    )(h, x, wh[0], wh[1], wh[2], wx[0], wx[1], wx[2], b, g, be,
      cw2, cb2, mw, mb)


def _h4_body(y_ref, z_ref, o_ref):
    prod = y_ref[...] * z_ref[...]
    m = jnp.sum(prod, axis=0, keepdims=True) / prod.shape[0]
    o_ref[...] = jax.nn.sigmoid(m)


def _h4(y, z):
    return pl.pallas_call(
        _h4_body,
        out_shape=jax.ShapeDtypeStruct((1, 1), jnp.float32),
    )(y, z)


def kernel(x, edge_index, edge_attr, ggnn_w, gru_w_ih, gru_w_hh, gru_b_ih,
           gru_b_hh, conv1_w, conv1_b, conv2_w, conv2_b, conv1c_w, conv1c_b,
           conv2c_w, conv2c_b, bn_g, bn_b, bnc_g, bnc_b, mlpy_w, mlpy_b,
           mlpz_w, mlpz_b):
    n, d = x.shape
    e = edge_attr.shape[0]
    c = conv1c_w.shape[0]

    src = edge_index[0].astype(jnp.int32)
    dst = edge_index[1].astype(jnp.int32)

    sc_msg = _make_sc_msg(n, d, e)

    wih_t = gru_w_ih.T
    whh_t = gru_w_hh.T
    bih = gru_b_ih.reshape(1, 3 * d)
    bhh = gru_b_hh.reshape(1, 3 * d)

    h = x
    m = _matmul(h, ggnn_w[0])
    for i in range(_T):
        agg2 = sc_msg(m, src, dst, edge_attr)
        wnext = ggnn_w[i + 1] if i + 1 < _T else ggnn_w[0]
        h, m = _gru(agg2, h, wih_t, whh_t, bih, bhh, wnext)

    w1k = [conv1_w[:, :, t].T for t in range(3)]
    y3 = _hy(h, w1k[0], w1k[1], w1k[2], conv1_b.reshape(1, d),
             bn_g.reshape(1, d), bn_b.reshape(1, d), conv2_w[:, :, 0].T,
             conv2_b.reshape(1, d), mlpy_w.T,
             mlpy_b.reshape(1, 1))

    whk = [conv1c_w[:, 0:d, t].T for t in range(3)]
    wxk = [conv1c_w[:, d:2 * d, t].T for t in range(3)]
    z3 = _hz(h, x, whk, wxk, conv1c_b.reshape(1, c),
             bnc_g.reshape(1, c), bnc_b.reshape(1, c), conv2c_w[:, :, 0].T,
             conv2c_b.reshape(1, c), mlpz_w.T,
             mlpz_b.reshape(1, 1))

    out = _h4(y3, z3)
    return out.reshape((1,))
